# baseline (speedup 1.0000x reference)
import functools

import jax
import jax.numpy as jnp
from jax import lax
from jax.experimental import pallas as pl
from jax.experimental.pallas import tpu as pltpu
from jax.experimental.pallas import tpu_sc as plsc

_VOCAB = 8192
_D = 8192
_SPLIT = 2
_DSUB = _D // _SPLIT
_B, _S = 8, 1024
_NTOK = _B * _S
_R = _NTOK * _SPLIT
_NC, _NS = 2, 16
_NW = _NC * _NS
_PER_W = _R // _NW
_C = 8
_NCH = _PER_W // _C


def _body(idx_hbm, table_hbm, out_hbm, idx_v, b0, b1, gs0, gs1, os0, os1):
    wid = lax.axis_index("s") * _NC + lax.axis_index("c")
    base = wid * _PER_W
    pltpu.sync_copy(idx_hbm.at[wid], idx_v)

    def g_start(i, buf, sem):
        pltpu.async_copy(table_hbm.at[idx_v.at[i]], buf, sem)

    def g_wait(i, buf, sem):
        pltpu.make_async_copy(table_hbm.at[idx_v.at[i]], buf, sem).wait()

    def o_start(i, buf, sem):
        pltpu.async_copy(buf, out_hbm.at[pl.ds(base + i * _C, _C)], sem)

    def o_wait(i, buf, sem):
        pltpu.make_async_copy(buf, out_hbm.at[pl.ds(base + i * _C, _C)], sem).wait()

    g_start(0, b0, gs0)
    g_start(1, b1, gs1)

    def step(k, carry):
        i0 = 2 * k
        i1 = i0 + 1
        g_wait(i0, b0, gs0)
        o_start(i0, b0, os0)
        g_wait(i1, b1, gs1)
        o_start(i1, b1, os1)
        o_wait(i0, b0, os0)
        g_start(i0 + 2, b0, gs0)
        o_wait(i1, b1, os1)
        g_start(i1 + 2, b1, gs1)
        return carry

    lax.fori_loop(0, _NCH // 2 - 1, step, 0)

    i0, i1 = _NCH - 2, _NCH - 1
    g_wait(i0, b0, gs0)
    o_start(i0, b0, os0)
    g_wait(i1, b1, gs1)
    o_start(i1, b1, os1)
    o_wait(i0, b0, os0)
    o_wait(i1, b1, os1)


_gather = functools.partial(
    pl.kernel,
    out_type=jax.ShapeDtypeStruct((_R, _DSUB), jnp.float32),
    mesh=plsc.VectorSubcoreMesh(core_axis_name="c", subcore_axis_name="s"),
    scratch_types=[
        pltpu.VMEM((_NCH, _C), jnp.int32),
        pltpu.VMEM((_C, _DSUB), jnp.float32),
        pltpu.VMEM((_C, _DSUB), jnp.float32),
        pltpu.SemaphoreType.DMA,
        pltpu.SemaphoreType.DMA,
        pltpu.SemaphoreType.DMA,
        pltpu.SemaphoreType.DMA,
    ],
)(_body)


def kernel(x, table):
    x32 = x.reshape(-1).astype(jnp.int32)
    idx2 = x32[:, None] * _SPLIT + jnp.arange(_SPLIT, dtype=jnp.int32)[None, :]
    idx3 = idx2.reshape(_NW, _NCH, _C)
    table2 = table.reshape(_VOCAB * _SPLIT, _DSUB)
    out2 = _gather(idx3, table2)
    return out2.reshape(_B, _S, _D)


# speedup vs baseline: 1.0028x; 1.0028x over previous
import functools

import jax
import jax.numpy as jnp
from jax import lax
from jax.experimental import pallas as pl
from jax.experimental.pallas import tpu as pltpu
from jax.experimental.pallas import tpu_sc as plsc

_VOCAB = 8192
_D = 8192
_SPLIT = 2
_DSUB = _D // _SPLIT
_B, _S = 8, 1024
_NTOK = _B * _S
_R = _NTOK * _SPLIT
_NC, _NS = 2, 16
_NW = _NC * _NS
_PER_W = _R // _NW
_C = 8
_NCH = _PER_W // _C


_NBUF = 3


def _body(idx_hbm, table_hbm, out_hbm, idx_v, b0, b1, b2,
          gs0, gs1, gs2, os0, os1, os2):
    wid = lax.axis_index("s") * _NC + lax.axis_index("c")
    base = wid * _PER_W
    pltpu.sync_copy(idx_hbm.at[wid], idx_v)

    bufs = (b0, b1, b2)
    gsems = (gs0, gs1, gs2)
    osems = (os0, os1, os2)

    def g_start(i):
        s = i % _NBUF
        pltpu.async_copy(table_hbm.at[idx_v.at[i]], bufs[s], gsems[s])

    def g_wait(i):
        s = i % _NBUF
        pltpu.make_async_copy(table_hbm.at[idx_v.at[i]], bufs[s], gsems[s]).wait()

    def o_start(i):
        s = i % _NBUF
        pltpu.async_copy(bufs[s], out_hbm.at[pl.ds(base + i * _C, _C)], osems[s])

    def o_wait(i):
        s = i % _NBUF
        pltpu.make_async_copy(
            bufs[s], out_hbm.at[pl.ds(base + i * _C, _C)], osems[s]).wait()

    g_start(0)
    g_start(1)
    for k in range(_NCH):
        g_wait(k)
        o_start(k)
        if k >= 1 and k + 1 < _NCH:
            o_wait(k - 1)
        if k + 2 < _NCH:
            g_start(k + 2)
    o_wait(_NCH - 2)
    o_wait(_NCH - 1)


_gather = functools.partial(
    pl.kernel,
    out_type=jax.ShapeDtypeStruct((_R, _DSUB), jnp.float32),
    mesh=plsc.VectorSubcoreMesh(core_axis_name="c", subcore_axis_name="s"),
    scratch_types=[
        pltpu.VMEM((_NCH, _C), jnp.int32),
        pltpu.VMEM((_C, _DSUB), jnp.float32),
        pltpu.VMEM((_C, _DSUB), jnp.float32),
        pltpu.VMEM((_C, _DSUB), jnp.float32),
        pltpu.SemaphoreType.DMA,
        pltpu.SemaphoreType.DMA,
        pltpu.SemaphoreType.DMA,
        pltpu.SemaphoreType.DMA,
        pltpu.SemaphoreType.DMA,
        pltpu.SemaphoreType.DMA,
    ],
)(_body)


def kernel(x, table):
    x32 = x.reshape(-1).astype(jnp.int32)
    idx2 = x32[:, None] * _SPLIT + jnp.arange(_SPLIT, dtype=jnp.int32)[None, :]
    idx3 = idx2.reshape(_NW, _NCH, _C)
    table2 = table.reshape(_VOCAB * _SPLIT, _DSUB)
    out2 = _gather(idx3, table2)
    return out2.reshape(_B, _S, _D)
